# baseline (speedup 1.0000x reference)
import functools

import jax
import jax.numpy as jnp
from jax import lax
from jax.experimental import pallas as pl
from jax.experimental.pallas import tpu as pltpu
from jax.experimental.pallas import tpu_sc as plsc

N = 10000
E = 320000
D = 128
FH = D // 2

NC = 2
NS = 16
CHUNK = 128
EPAD = 327680
NCHUNKS = EPAD // CHUNK
CPT = NCHUNKS // NS
NPAD = 12800
ZPT = NPAD // NS
ZROWS = 80
DGW = 16

_mesh = plsc.VectorSubcoreMesh(core_axis_name="c", subcore_axis_name="s")


def _sc_agg_body(with_deg, *refs):
    if with_deg:
        (src_hbm, dst_hbm, x_hbm, agg_hbm, deg_hbm,
         srcb, dstb, rows0, rows1, ones, zbuf, zbufd, aggs, degs) = refs
    else:
        (src_hbm, dst_hbm, x_hbm, agg_hbm,
         srcb, dstb, rows0, rows1, zbuf, aggs) = refs

    cid = lax.axis_index("c")
    sid = lax.axis_index("s")

    zvec = jnp.zeros((16,), jnp.float32)

    @pl.loop(0, ZROWS)
    def _(r):
        @pl.loop(0, FH, step=16)
        def _(c):
            zbuf[r, pl.ds(c, 16)] = zvec

    if with_deg:
        lanes = lax.iota(jnp.int32, 16)
        onerow = jnp.where(lanes == 0, 1.0, 0.0).astype(jnp.float32)

        @pl.loop(0, CHUNK)
        def _(r):
            ones[r, pl.ds(0, 16)] = onerow

        @pl.loop(0, ZROWS)
        def _(r):
            zbufd[r, pl.ds(0, 16)] = zvec

    zbase = sid * ZPT

    @pl.loop(0, ZPT, step=ZROWS)
    def _(r):
        pltpu.sync_copy(zbuf, aggs.at[pl.ds(zbase + r, ZROWS)])

    if with_deg:
        @pl.loop(0, ZPT, step=ZROWS)
        def _(r):
            pltpu.sync_copy(zbufd, degs.at[pl.ds(zbase + r, ZROWS)])

    plsc.subcore_barrier()

    c0 = sid * CPT
    pltpu.sync_copy(src_hbm.at[pl.ds(cid * NCHUNKS + c0, CPT)], srcb)
    pltpu.sync_copy(dst_hbm.at[pl.ds(c0, CPT)], dstb)

    bufs = (rows0, rows1)

    @functools.partial(pl.run_scoped,
                       gsem=pltpu.SemaphoreType.DMA(()),
                       ssem=pltpu.SemaphoreType.DMA(()))
    def _(gsem, ssem):
        def _gather(j, b):
            return pltpu.make_async_copy(x_hbm.at[srcb.at[j]],
                                         bufs[b], gsem)

        def _scat(j, b):
            return pltpu.make_async_copy(bufs[b], aggs.at[dstb.at[j]],
                                         ssem)

        _gather(0, 0).start()

        @pl.loop(0, CPT, step=2)
        def _(j0):
            for b in range(2):
                j = j0 + b
                nb = 1 - b
                _gather(j, b).wait()
                _scat(j, b).start(add=True)
                if with_deg:
                    pltpu.sync_copy(ones, degs.at[dstb.at[j]], add=True)

                @pl.when(j + 1 < CPT)
                def _():
                    @pl.when(j >= 1)
                    def _():
                        _scat(j - 1, nb).wait()

                    _gather(j + 1, nb).start()

        for b in range(2):
            j = CPT - 2 + b
            _scat(j, b).wait()

    plsc.subcore_barrier()

    obase = sid * ZPT
    pltpu.sync_copy(aggs.at[pl.ds(obase, ZPT)],
                    agg_hbm.at[pl.ds(cid * NPAD + obase, ZPT)])
    if with_deg:
        pltpu.sync_copy(degs.at[pl.ds(obase, ZPT)],
                        deg_hbm.at[pl.ds(cid * NPAD + obase, ZPT)])


def _make_sc_agg(with_deg):
    out_type = [jax.ShapeDtypeStruct((NC * NPAD, FH), jnp.float32)]
    scratch = [
        pltpu.VMEM((CPT, CHUNK), jnp.int32),
        pltpu.VMEM((CPT, CHUNK), jnp.int32),
        pltpu.VMEM((CHUNK, FH), jnp.float32),
        pltpu.VMEM((CHUNK, FH), jnp.float32),
    ]
    if with_deg:
        out_type.append(jax.ShapeDtypeStruct((NC * NPAD, DGW), jnp.float32))
        scratch.append(pltpu.VMEM((CHUNK, DGW), jnp.float32))
    scratch.append(pltpu.VMEM((ZROWS, FH), jnp.float32))
    if with_deg:
        scratch.append(pltpu.VMEM((ZROWS, DGW), jnp.float32))
    scratch.append(pltpu.VMEM_SHARED((NPAD, FH), jnp.float32))
    if with_deg:
        scratch.append(pltpu.VMEM_SHARED((NPAD, DGW), jnp.float32))
    return pl.kernel(
        functools.partial(_sc_agg_body, with_deg),
        out_type=out_type,
        mesh=_mesh,
        scratch_types=scratch,
        compiler_params=pltpu.CompilerParams(use_tc_tiling_on_sc=False),
    )


_sc_agg_deg = _make_sc_agg(True)
_sc_agg = _make_sc_agg(False)


BR = 400
GRID = N // BR
POFF = NPAD // BR


def _dense1_body(p0, p1, g0, x, w1lt, b1l, w1rt, o0, o1):
    deg = jnp.maximum(g0[:, 0:1], 1.0)
    agg = jnp.concatenate([p0[...], p1[...]], axis=1) / deg
    hi = jax.lax.Precision.HIGHEST
    acc = jnp.dot(agg, w1lt[...], preferred_element_type=jnp.float32,
                  precision=hi)
    acc += jnp.dot(x[...], w1rt[...], preferred_element_type=jnp.float32,
                   precision=hi)
    h = jnp.maximum(acc + b1l[...], 0.0)
    o0[...] = h[:, :FH]
    o1[...] = h[:, FH:]


def _dense2_body(p0, p1, g0, h0, h1, w2lt, b2l, w2rt, wm1t, bm1, wm2t,
                 bm2, o):
    deg = jnp.maximum(g0[:, 0:1], 1.0)
    agg = jnp.concatenate([p0[...], p1[...]], axis=1) / deg
    h = jnp.concatenate([h0[...], h1[...]], axis=1)
    hi = jax.lax.Precision.HIGHEST
    acc = jnp.dot(agg, w2lt[...], preferred_element_type=jnp.float32,
                  precision=hi)
    acc += jnp.dot(h, w2rt[...], preferred_element_type=jnp.float32,
                   precision=hi)
    t = jnp.maximum(acc + b2l[...], 0.0)
    u = jnp.maximum(jnp.dot(t, wm1t[...], preferred_element_type=jnp.float32,
                            precision=hi) + bm1[...], 0.0)
    o[...] = jnp.dot(u, wm2t[...], preferred_element_type=jnp.float32,
                     precision=hi) + bm2[...]


def _row_spec(shape, off=0):
    return pl.BlockSpec((BR,) + shape[1:], lambda i, _o=off: (i + _o, 0))


def _full_spec(shape):
    return pl.BlockSpec(shape, lambda i: (0, 0))


_dense1 = pl.pallas_call(
    _dense1_body,
    grid=(GRID,),
    in_specs=[
        _row_spec((BR, FH)), _row_spec((BR, FH), POFF),
        _row_spec((BR, DGW)),
        _row_spec((BR, D)),
        _full_spec((D, D)), _full_spec((1, D)), _full_spec((D, D)),
    ],
    out_specs=[_row_spec((BR, FH)), _row_spec((BR, FH))],
    out_shape=[jax.ShapeDtypeStruct((N, FH), jnp.float32),
               jax.ShapeDtypeStruct((N, FH), jnp.float32)],
)

_dense2 = pl.pallas_call(
    _dense2_body,
    grid=(GRID,),
    in_specs=[
        _row_spec((BR, FH)), _row_spec((BR, FH), POFF),
        _row_spec((BR, DGW)),
        _row_spec((BR, FH)), _row_spec((BR, FH)),
        _full_spec((D, D)), _full_spec((1, D)), _full_spec((D, D)),
        _full_spec((D, D)), _full_spec((1, D)),
        _full_spec((D, 1)), _full_spec((1, 1)),
    ],
    out_specs=_row_spec((BR, 1)),
    out_shape=jax.ShapeDtypeStruct((N, 1), jnp.float32),
)


def kernel(x, edge_index, W1l, b1l, W1r, W2l, b2l, W2r, Wm1, bm1, Wm2, bm2):
    src = edge_index[0]
    dst = edge_index[1]
    npad = EPAD - E
    pad_src = jnp.arange(npad, dtype=jnp.int32) % N
    pad_dst = N + (jnp.arange(npad, dtype=jnp.int32) % (NPAD - N))
    srcp = jnp.concatenate([src, pad_src])
    src2 = jnp.concatenate([srcp, srcp + N]).reshape(NC * NCHUNKS, CHUNK)
    dst2 = jnp.concatenate([dst, pad_dst]).reshape(NCHUNKS, CHUNK)

    xt = jnp.concatenate([x[:, :FH], x[:, FH:]], axis=0)
    aggp1, degp = _sc_agg_deg(src2, dst2, xt)
    h0, h1 = _dense1(aggp1, aggp1, degp, x,
                     W1l.T, b1l.reshape(1, D), W1r.T)
    ht = jnp.concatenate([h0, h1], axis=0)
    res = _sc_agg(src2, dst2, ht)
    aggp2 = res[0] if isinstance(res, (list, tuple)) else res
    out = _dense2(aggp2, aggp2, degp, h0, h1,
                  W2l.T, b2l.reshape(1, D), W2r.T,
                  Wm1.T, bm1.reshape(1, D), Wm2.T, bm2.reshape(1, 1))
    return out.reshape(N)


# speedup vs baseline: 1.1121x; 1.1121x over previous
import functools

import jax
import jax.numpy as jnp
from jax import lax
from jax.experimental import pallas as pl
from jax.experimental.pallas import tpu as pltpu
from jax.experimental.pallas import tpu_sc as plsc

N = 10000
E = 320000
D = 128
FH = D // 2

NC = 2
NS = 16
CHUNK = 128
EPAD = 327680
NCHUNKS = EPAD // CHUNK
CPT = NCHUNKS // NS
NPAD = 12800
ZPT = NPAD // NS
ZROWS = 80
DGW = 16

_mesh = plsc.VectorSubcoreMesh(core_axis_name="c", subcore_axis_name="s")


def _sc_agg_body(with_deg, *refs):
    if with_deg:
        (src_hbm, dst_hbm, x_hbm, agg_hbm, deg_hbm,
         srcb, dstb, rows0, rows1, ones, zbuf, zbufd, aggs, degs) = refs
    else:
        (src_hbm, dst_hbm, x_hbm, agg_hbm,
         srcb, dstb, rows0, rows1, zbuf, aggs) = refs

    cid = lax.axis_index("c")
    sid = lax.axis_index("s")

    zvec = jnp.zeros((16,), jnp.float32)

    @pl.loop(0, ZROWS)
    def _(r):
        @pl.loop(0, FH, step=16)
        def _(c):
            zbuf[r, pl.ds(c, 16)] = zvec

    if with_deg:
        lanes = lax.iota(jnp.int32, 16)
        onerow = jnp.where(lanes == 0, 1.0, 0.0).astype(jnp.float32)

        @pl.loop(0, CHUNK)
        def _(r):
            ones[r, pl.ds(0, 16)] = onerow

        @pl.loop(0, ZROWS)
        def _(r):
            zbufd[r, pl.ds(0, 16)] = zvec

    zbase = sid * ZPT

    @pl.loop(0, ZPT, step=ZROWS)
    def _(r):
        pltpu.sync_copy(zbuf, aggs.at[pl.ds(zbase + r, ZROWS)])

    if with_deg:
        @pl.loop(0, ZPT, step=ZROWS)
        def _(r):
            pltpu.sync_copy(zbufd, degs.at[pl.ds(zbase + r, ZROWS)])

    plsc.subcore_barrier()

    c0 = sid * CPT
    pltpu.sync_copy(src_hbm.at[pl.ds(cid * NCHUNKS + c0, CPT)], srcb)
    pltpu.sync_copy(dst_hbm.at[pl.ds(c0, CPT)], dstb)

    bufs = (rows0, rows1)

    @functools.partial(pl.run_scoped,
                       gsem=pltpu.SemaphoreType.DMA(()),
                       ssem=pltpu.SemaphoreType.DMA(()))
    def _(gsem, ssem):
        def _gather(j, b):
            return pltpu.make_async_copy(x_hbm.at[srcb.at[j]],
                                         bufs[b], gsem)

        def _scat(j, b):
            return pltpu.make_async_copy(bufs[b], aggs.at[dstb.at[j]],
                                         ssem)

        _gather(0, 0).start()

        @pl.loop(0, CPT, step=2)
        def _(j0):
            for b in range(2):
                j = j0 + b
                _gather(j, b).wait()

                @pl.when(j + 1 < CPT)
                def _():
                    _gather(j + 1, 1 - b).start()

                pltpu.sync_copy(bufs[b], aggs.at[dstb.at[j]], add=True)
                if with_deg:
                    pltpu.sync_copy(ones, degs.at[dstb.at[j]], add=True)

    plsc.subcore_barrier()

    obase = sid * ZPT
    pltpu.sync_copy(aggs.at[pl.ds(obase, ZPT)],
                    agg_hbm.at[pl.ds(cid * NPAD + obase, ZPT)])
    if with_deg:
        pltpu.sync_copy(degs.at[pl.ds(obase, ZPT)],
                        deg_hbm.at[pl.ds(cid * NPAD + obase, ZPT)])


def _make_sc_agg(with_deg):
    out_type = [jax.ShapeDtypeStruct((NC * NPAD, FH), jnp.float32)]
    scratch = [
        pltpu.VMEM((CPT, CHUNK), jnp.int32),
        pltpu.VMEM((CPT, CHUNK), jnp.int32),
        pltpu.VMEM((CHUNK, FH), jnp.float32),
        pltpu.VMEM((CHUNK, FH), jnp.float32),
    ]
    if with_deg:
        out_type.append(jax.ShapeDtypeStruct((NC * NPAD, DGW), jnp.float32))
        scratch.append(pltpu.VMEM((CHUNK, DGW), jnp.float32))
    scratch.append(pltpu.VMEM((ZROWS, FH), jnp.float32))
    if with_deg:
        scratch.append(pltpu.VMEM((ZROWS, DGW), jnp.float32))
    scratch.append(pltpu.VMEM_SHARED((NPAD, FH), jnp.float32))
    if with_deg:
        scratch.append(pltpu.VMEM_SHARED((NPAD, DGW), jnp.float32))
    return pl.kernel(
        functools.partial(_sc_agg_body, with_deg),
        out_type=out_type,
        mesh=_mesh,
        scratch_types=scratch,
        compiler_params=pltpu.CompilerParams(use_tc_tiling_on_sc=False),
    )


_sc_agg_deg = _make_sc_agg(True)
_sc_agg = _make_sc_agg(False)


BR = 400
GRID = N // BR
POFF = NPAD // BR


def _dense1_body(p0, p1, g0, x, w1lt, b1l, w1rt, o0, o1):
    deg = jnp.maximum(g0[:, 0:1], 1.0)
    agg = jnp.concatenate([p0[...], p1[...]], axis=1) / deg
    hi = jax.lax.Precision.DEFAULT
    acc = jnp.dot(agg, w1lt[...], preferred_element_type=jnp.float32,
                  precision=hi)
    acc += jnp.dot(x[...], w1rt[...], preferred_element_type=jnp.float32,
                   precision=hi)
    h = jnp.maximum(acc + b1l[...], 0.0)
    o0[...] = h[:, :FH]
    o1[...] = h[:, FH:]


def _dense2_body(p0, p1, g0, h0, h1, w2lt, b2l, w2rt, wm1t, bm1, wm2t,
                 bm2, o):
    deg = jnp.maximum(g0[:, 0:1], 1.0)
    agg = jnp.concatenate([p0[...], p1[...]], axis=1) / deg
    h = jnp.concatenate([h0[...], h1[...]], axis=1)
    hi = jax.lax.Precision.DEFAULT
    acc = jnp.dot(agg, w2lt[...], preferred_element_type=jnp.float32,
                  precision=hi)
    acc += jnp.dot(h, w2rt[...], preferred_element_type=jnp.float32,
                   precision=hi)
    t = jnp.maximum(acc + b2l[...], 0.0)
    u = jnp.maximum(jnp.dot(t, wm1t[...], preferred_element_type=jnp.float32,
                            precision=hi) + bm1[...], 0.0)
    o[...] = jnp.dot(u, wm2t[...], preferred_element_type=jnp.float32,
                     precision=hi) + bm2[...]


def _row_spec(shape, off=0):
    return pl.BlockSpec((BR,) + shape[1:], lambda i, _o=off: (i + _o, 0))


def _full_spec(shape):
    return pl.BlockSpec(shape, lambda i: (0, 0))


_dense1 = pl.pallas_call(
    _dense1_body,
    grid=(GRID,),
    in_specs=[
        _row_spec((BR, FH)), _row_spec((BR, FH), POFF),
        _row_spec((BR, DGW)),
        _row_spec((BR, D)),
        _full_spec((D, D)), _full_spec((1, D)), _full_spec((D, D)),
    ],
    out_specs=[_row_spec((BR, FH)), _row_spec((BR, FH))],
    out_shape=[jax.ShapeDtypeStruct((N, FH), jnp.float32),
               jax.ShapeDtypeStruct((N, FH), jnp.float32)],
)

_dense2 = pl.pallas_call(
    _dense2_body,
    grid=(GRID,),
    in_specs=[
        _row_spec((BR, FH)), _row_spec((BR, FH), POFF),
        _row_spec((BR, DGW)),
        _row_spec((BR, FH)), _row_spec((BR, FH)),
        _full_spec((D, D)), _full_spec((1, D)), _full_spec((D, D)),
        _full_spec((D, D)), _full_spec((1, D)),
        _full_spec((D, 1)), _full_spec((1, 1)),
    ],
    out_specs=_row_spec((BR, 1)),
    out_shape=jax.ShapeDtypeStruct((N, 1), jnp.float32),
)


def kernel(x, edge_index, W1l, b1l, W1r, W2l, b2l, W2r, Wm1, bm1, Wm2, bm2):
    src = edge_index[0]
    dst = edge_index[1]
    npad = EPAD - E
    pad_src = jnp.arange(npad, dtype=jnp.int32) % N
    pad_dst = N + (jnp.arange(npad, dtype=jnp.int32) % (NPAD - N))
    srcp = jnp.concatenate([src, pad_src])
    src2 = jnp.concatenate([srcp, srcp + N]).reshape(NC * NCHUNKS, CHUNK)
    dst2 = jnp.concatenate([dst, pad_dst]).reshape(NCHUNKS, CHUNK)

    xt = jnp.concatenate([x[:, :FH], x[:, FH:]], axis=0)
    aggp1, degp = _sc_agg_deg(src2, dst2, xt)
    h0, h1 = _dense1(aggp1, aggp1, degp, x,
                     W1l.T, b1l.reshape(1, D), W1r.T)
    ht = jnp.concatenate([h0, h1], axis=0)
    res = _sc_agg(src2, dst2, ht)
    aggp2 = res[0] if isinstance(res, (list, tuple)) else res
    out = _dense2(aggp2, aggp2, degp, h0, h1,
                  W2l.T, b2l.reshape(1, D), W2r.T,
                  Wm1.T, bm1.reshape(1, D), Wm2.T, bm2.reshape(1, 1))
    return out.reshape(N)
